# baseline (speedup 1.0000x reference)
import functools

import jax
import jax.numpy as jnp
from jax import lax
from jax.experimental import pallas as pl
from jax.experimental.pallas import tpu as pltpu
from jax.experimental.pallas import tpu_sc as plsc

PAD = 1
HID = 768
EPS = 1e-12
MAXITEM = 32

NC = 2
NS = 16
NW = NC * NS
LANES = 16
NVH = HID // LANES


def _posid_body(ids_ref, out_ref):
    ids = ids_ref[...]
    m = (ids != PAD).astype(jnp.int32)
    acc = m
    s = ids.shape[1]
    k = 1
    while k < s:
        shifted = jnp.concatenate(
            [jnp.zeros(ids.shape[:1] + (k,), jnp.int32), acc[:, :-k]], axis=1
        )
        acc = acc + shifted
        k *= 2
    out_ref[...] = acc * m + PAD


def _sc_body(tpw, chunk, word_hbm, pos_hbm, idw_hbm, idp_hbm, out_hbm,
             idw_all, idp_all, bw0, bp0, bw1, bp1, sem, wsem):
    wid = lax.axis_index("s") * NC + lax.axis_index("c")
    base = wid * tpw
    nchunks = tpw // chunk
    pltpu.sync_copy(idw_hbm.at[pl.ds(base, tpw)], idw_all)
    pltpu.sync_copy(idp_hbm.at[pl.ds(base, tpw)], idp_all)

    bufs = [(bw0, bp0), (bw1, bp1)]

    def issue(g):
        bw, bp = bufs[g % 2]
        sl = pl.ds(g * chunk, chunk)
        cw = pltpu.async_copy(word_hbm.at[idw_all.at[sl]], bw, sem)
        cp = pltpu.async_copy(pos_hbm.at[idp_all.at[sl]], bp, sem)
        return cw, cp

    pend = issue(0)
    wpend = [None, None]
    for g in range(nchunks):
        pend[0].wait()
        pend[1].wait()
        if g + 1 < nchunks:
            if wpend[(g + 1) % 2] is not None:
                wpend[(g + 1) % 2].wait()
                wpend[(g + 1) % 2] = None
            pend = issue(g + 1)
        bw, bp = bufs[g % 2]

        def add_token(t, _, bw=bw, bp=bp):
            for i in range(NVH):
                sl = pl.ds(i * LANES, LANES)
                plsc.addupdate(bw.at[t, sl], bp[t, sl])
            return 0

        lax.fori_loop(0, chunk, add_token, 0)
        wpend[g % 2] = pltpu.async_copy(
            bw, out_hbm.at[pl.ds(base + g * chunk, chunk)], wsem)
    for w in wpend:
        if w is not None:
            w.wait()


def _ln_body(ids_ref, x_ref, item_ref, w_ref, b_ref, o_ref):
    x = x_ref[...]
    ids = ids_ref[...]
    onehot = (ids == lax.broadcasted_iota(
        jnp.int32, (ids.shape[0], MAXITEM), 1)).astype(jnp.float32)
    x = x + jnp.dot(onehot, item_ref[...], preferred_element_type=jnp.float32,
                    precision=lax.Precision.HIGHEST)
    mu = jnp.mean(x, axis=-1, keepdims=True)
    var = jnp.mean(jnp.square(x - mu), axis=-1, keepdims=True)
    o_ref[...] = (x - mu) / jnp.sqrt(var + EPS) * w_ref[...] + b_ref[...]


def kernel(input_ids, item_position_ids, word_embeddings, position_embeddings,
           item_position_embeddings, ln_weight, ln_bias):
    b, s = input_ids.shape
    n = b * s
    half = n // 2
    tpw = half // NW
    chunk = 32

    position_ids = pl.pallas_call(
        _posid_body,
        out_shape=jax.ShapeDtypeStruct((b, s), jnp.int32),
    )(input_ids.astype(jnp.int32))

    mesh = plsc.VectorSubcoreMesh(core_axis_name="c", subcore_axis_name="s")
    sc = pl.kernel(
        functools.partial(_sc_body, tpw, chunk),
        out_type=jax.ShapeDtypeStruct((half, HID), jnp.float32),
        mesh=mesh,
        scratch_types=[
            pltpu.VMEM((tpw,), jnp.int32),
            pltpu.VMEM((tpw,), jnp.int32),
            pltpu.VMEM((chunk, HID), jnp.float32),
            pltpu.VMEM((chunk, HID), jnp.float32),
            pltpu.VMEM((chunk, HID), jnp.float32),
            pltpu.VMEM((chunk, HID), jnp.float32),
            pltpu.SemaphoreType.DMA,
            pltpu.SemaphoreType.DMA,
        ],
    )

    idw = input_ids.reshape(n).astype(jnp.int32)
    idp = position_ids.reshape(n)
    idi = item_position_ids.reshape(n, 1).astype(jnp.int32)

    tb = 512
    nblk = half // tb
    ln = functools.partial(
        pl.pallas_call,
        _ln_body,
        grid=(nblk,),
        in_specs=[
            pl.BlockSpec((tb, 1), lambda i: (i, 0)),
            pl.BlockSpec((tb, HID), lambda i: (i, 0)),
            pl.BlockSpec((MAXITEM, HID), lambda i: (0, 0)),
            pl.BlockSpec((HID,), lambda i: (0,)),
            pl.BlockSpec((HID,), lambda i: (0,)),
        ],
        out_specs=pl.BlockSpec((tb, HID), lambda i: (i, 0)),
        out_shape=jax.ShapeDtypeStruct((half, HID), jnp.float32),
    )()

    summed0 = sc(word_embeddings, position_embeddings, idw[:half], idp[:half])
    summed1 = sc(word_embeddings, position_embeddings, idw[half:], idp[half:])
    out0 = ln(idi[:half], summed0, item_position_embeddings, ln_weight, ln_bias)
    out1 = ln(idi[half:], summed1, item_position_embeddings, ln_weight, ln_bias)
    out = jnp.concatenate([out0, out1], axis=0)
    return out.reshape(b, s, HID)


# speedup vs baseline: 1.1487x; 1.1487x over previous
import functools

import jax
import jax.numpy as jnp
from jax import lax
from jax.experimental import pallas as pl
from jax.experimental.pallas import tpu as pltpu
from jax.experimental.pallas import tpu_sc as plsc

PAD = 1
HID = 768
EPS = 1e-12
MAXITEM = 32

NC = 2
NS = 16
NW = NC * NS
LANES = 16
NVH = HID // LANES


def _posid_body(ids_ref, out_ref):
    ids = ids_ref[...]
    m = (ids != PAD).astype(jnp.int32)
    acc = m
    s = ids.shape[1]
    k = 1
    while k < s:
        shifted = jnp.concatenate(
            [jnp.zeros(ids.shape[:1] + (k,), jnp.int32), acc[:, :-k]], axis=1
        )
        acc = acc + shifted
        k *= 2
    out_ref[...] = acc * m + PAD


def _sc_body(tpw, chunk, word_hbm, pos_hbm, idw_hbm, idp_hbm, out_hbm,
             idw_all, idp_all, bw0, bp0, bw1, bp1, sem, wsem):
    wid = lax.axis_index("s") * NC + lax.axis_index("c")
    base = wid * tpw
    nchunks = tpw // chunk
    pltpu.sync_copy(idw_hbm.at[pl.ds(base, tpw)], idw_all)
    pltpu.sync_copy(idp_hbm.at[pl.ds(base, tpw)], idp_all)

    bufs = [(bw0, bp0), (bw1, bp1)]

    def issue(g):
        bw, bp = bufs[g % 2]
        sl = pl.ds(g * chunk, chunk)
        cw = pltpu.async_copy(word_hbm.at[idw_all.at[sl]], bw, sem)
        cp = pltpu.async_copy(pos_hbm.at[idp_all.at[sl]], bp, sem)
        return cw, cp

    pend = issue(0)
    wpend = [None, None]
    for g in range(nchunks):
        pend[0].wait()
        pend[1].wait()
        if g + 1 < nchunks:
            if wpend[(g + 1) % 2] is not None:
                wpend[(g + 1) % 2].wait()
                wpend[(g + 1) % 2] = None
            pend = issue(g + 1)
        bw, bp = bufs[g % 2]

        def add_token(t, _, bw=bw, bp=bp):
            for i in range(NVH):
                sl = pl.ds(i * LANES, LANES)
                plsc.addupdate(bw.at[t, sl], bp[t, sl])
            return 0

        lax.fori_loop(0, chunk, add_token, 0)
        wpend[g % 2] = pltpu.async_copy(
            bw, out_hbm.at[pl.ds(base + g * chunk, chunk)], wsem)
    for w in wpend:
        if w is not None:
            w.wait()


def _ln_body(ids_ref, x_ref, item_ref, w_ref, b_ref, o_ref):
    x = x_ref[...]
    ids = ids_ref[...]
    onehot = (ids == lax.broadcasted_iota(
        jnp.int32, (ids.shape[0], MAXITEM), 1)).astype(jnp.float32)
    x = x + jnp.dot(onehot, item_ref[...], preferred_element_type=jnp.float32,
                    precision=lax.Precision.HIGHEST)
    mu = jnp.mean(x, axis=-1, keepdims=True)
    var = jnp.mean(jnp.square(x - mu), axis=-1, keepdims=True)
    o_ref[...] = (x - mu) / jnp.sqrt(var + EPS) * w_ref[...] + b_ref[...]


def kernel(input_ids, item_position_ids, word_embeddings, position_embeddings,
           item_position_embeddings, ln_weight, ln_bias):
    b, s = input_ids.shape
    n = b * s
    tpw = n // NW
    chunk = 32

    position_ids = pl.pallas_call(
        _posid_body,
        out_shape=jax.ShapeDtypeStruct((b, s), jnp.int32),
    )(input_ids.astype(jnp.int32))

    mesh = plsc.VectorSubcoreMesh(core_axis_name="c", subcore_axis_name="s")
    sc = pl.kernel(
        functools.partial(_sc_body, tpw, chunk),
        out_type=jax.ShapeDtypeStruct((n, HID), jnp.float32),
        mesh=mesh,
        scratch_types=[
            pltpu.VMEM((tpw,), jnp.int32),
            pltpu.VMEM((tpw,), jnp.int32),
            pltpu.VMEM((chunk, HID), jnp.float32),
            pltpu.VMEM((chunk, HID), jnp.float32),
            pltpu.VMEM((chunk, HID), jnp.float32),
            pltpu.VMEM((chunk, HID), jnp.float32),
            pltpu.SemaphoreType.DMA,
            pltpu.SemaphoreType.DMA,
        ],
    )

    idw = input_ids.reshape(n).astype(jnp.int32)
    idp = position_ids.reshape(n)
    idi = item_position_ids.reshape(n, 1).astype(jnp.int32)

    summed = sc(word_embeddings, position_embeddings, idw, idp)

    tb = 512
    nblk = n // tb
    out = pl.pallas_call(
        _ln_body,
        grid=(nblk,),
        in_specs=[
            pl.BlockSpec((tb, 1), lambda i: (i, 0)),
            pl.BlockSpec((tb, HID), lambda i: (i, 0)),
            pl.BlockSpec((MAXITEM, HID), lambda i: (0, 0)),
            pl.BlockSpec((HID,), lambda i: (0,)),
            pl.BlockSpec((HID,), lambda i: (0,)),
        ],
        out_specs=pl.BlockSpec((tb, HID), lambda i: (i, 0)),
        out_shape=jax.ShapeDtypeStruct((n, HID), jnp.float32),
    )(idi, summed, item_position_embeddings, ln_weight, ln_bias)
    return out.reshape(b, s, HID)
